# baseline (speedup 1.0000x reference)
import functools

import jax
import jax.numpy as jnp
from jax.experimental import pallas as pl
from jax.experimental.pallas import tpu as pltpu

ROW_PENALTY = 10.0

_DC = 128


def _loss_kernel(x_ref, h_ref, c_ref, m_ref, out_ref,
                 cnt_a, b_a, p_a, q_a, r_a, mse_a):
    k = pl.program_id(0)
    nk = pl.num_programs(0)
    T = x_ref.shape[0]
    f32 = jnp.float32

    X = x_ref[...]
    H = h_ref[...]
    C = c_ref[...]
    Mc = m_ref[:, pl.ds(k * _DC, _DC)]

    mask = (Mc > 0).astype(f32)
    U = mask * H
    V = U * H
    resid = (X - H + C) * Mc

    dot_t = functools.partial(
        jax.lax.dot_general,
        dimension_numbers=(((1,), (1,)), ((), ())),
        preferred_element_type=f32,
    )

    cnt_p = dot_t(mask, mask)
    b_p = dot_t(U, mask)
    p_p = dot_t(V, mask)
    q_p = dot_t(U, U)
    r_p = dot_t(H, H)
    mse_p = jnp.sum(resid * resid)

    @pl.when(k == 0)
    def _init():
        cnt_a[...] = cnt_p
        b_a[...] = b_p
        p_a[...] = p_p
        q_a[...] = q_p
        r_a[...] = r_p
        mse_a[0] = mse_p

    @pl.when(k > 0)
    def _acc():
        cnt_a[...] += cnt_p
        b_a[...] += b_p
        p_a[...] += p_p
        q_a[...] += q_p
        r_a[...] += r_p
        mse_a[0] += mse_p

    @pl.when(k == nk - 1)
    def _finalize():
        cnt = cnt_a[...]
        B = b_a[...]
        P = p_a[...]
        Q = q_a[...]
        R = r_a[...]
        M = m_ref[...]

        S1 = B - B.T
        S2 = P + P.T - 2.0 * Q
        mean = S1 / jnp.maximum(cnt, 1.0)
        var = (S2 - S1 * mean) / jnp.maximum(cnt - 1.0, 1.0)

        d_iota = jax.lax.broadcasted_iota(jnp.int32, M.shape, 1)
        row_min = jnp.min(M, axis=1, keepdims=True)
        am = jnp.min(jnp.where(M == row_min, d_iota, M.shape[1]), axis=1,
                     keepdims=True)

        iota_r = jax.lax.broadcasted_iota(jnp.int32, (T, T), 0)
        iota_c = jax.lax.broadcasted_iota(jnp.int32, (T, T), 1)
        eye = iota_r == iota_c
        valid = (~eye) & (am != am.reshape(1, T))
        scores = jnp.where(valid, var, 9999.0)

        s_min = jnp.min(scores, axis=1, keepdims=True)
        min_row = jnp.min(jnp.where(scores == s_min, iota_c, T), axis=1,
                          keepdims=True)
        onehot = min_row == iota_c

        h2 = jnp.sum(jnp.where(eye, R, 0.0), axis=1, keepdims=True)
        norm2 = jnp.maximum(h2 + h2.reshape(1, T) - 2.0 * R, 0.0)
        sel = jnp.sum(jnp.where(onehot, norm2, 0.0), axis=1)
        row_loss = jnp.sum(jnp.sqrt(sel))

        out_ref[...] = jnp.reshape(mse_a[0] + ROW_PENALTY * row_loss, (1, 1))


def kernel(X, H, C, M, T):
    Tn, D = X.shape
    nk = D // _DC
    chunk = pl.BlockSpec((Tn, _DC), lambda k: (0, k))
    whole = pl.BlockSpec((Tn, D), lambda k: (0, 0))
    out = pl.pallas_call(
        _loss_kernel,
        grid=(nk,),
        in_specs=[chunk, chunk, chunk, whole],
        out_specs=pl.BlockSpec((1, 1), lambda k: (0, 0)),
        out_shape=jax.ShapeDtypeStruct((1, 1), jnp.float32),
        scratch_shapes=[pltpu.VMEM((Tn, Tn), jnp.float32)] * 5
        + [pltpu.SMEM((1,), jnp.float32)],
        compiler_params=pltpu.CompilerParams(
            dimension_semantics=("arbitrary",),
        ),
    )(X, H, C, M)
    return out[0, 0]


# speedup vs baseline: 1.3235x; 1.3235x over previous
import functools

import jax
import jax.numpy as jnp
from jax.experimental import pallas as pl
from jax.experimental.pallas import tpu as pltpu

ROW_PENALTY = 10.0

_DC = 256


def _loss_kernel(x_ref, h_ref, c_ref, m_ref, out_ref,
                 cnt_a, b_a, p_a, q_a, r_a, mse_a):
    k = pl.program_id(0)
    nk = pl.num_programs(0)
    T = x_ref.shape[0]
    f32 = jnp.float32

    X = x_ref[...]
    H = h_ref[...]
    C = c_ref[...]
    Mc = m_ref[:, pl.ds(k * _DC, _DC)]

    mask = (Mc > 0).astype(f32)
    U = mask * H
    V = U * H
    resid = (X - H + C) * Mc

    dot_t = functools.partial(
        jax.lax.dot_general,
        dimension_numbers=(((1,), (1,)), ((), ())),
        preferred_element_type=f32,
    )

    cnt_p = dot_t(mask, mask)
    b_p = dot_t(U, mask)
    p_p = dot_t(V, mask)
    q_p = dot_t(U, U)
    r_p = dot_t(H, H)
    mse_p = jnp.sum(resid * resid)

    @pl.when(k == 0)
    def _init():
        cnt_a[...] = cnt_p
        b_a[...] = b_p
        p_a[...] = p_p
        q_a[...] = q_p
        r_a[...] = r_p
        mse_a[0] = mse_p

    @pl.when(k > 0)
    def _acc():
        cnt_a[...] += cnt_p
        b_a[...] += b_p
        p_a[...] += p_p
        q_a[...] += q_p
        r_a[...] += r_p
        mse_a[0] += mse_p

    @pl.when(k == nk - 1)
    def _finalize():
        cnt = cnt_a[...]
        B = b_a[...]
        P = p_a[...]
        Q = q_a[...]
        R = r_a[...]
        M = m_ref[...]

        S1 = B - B.T
        S2 = P + P.T - 2.0 * Q
        mean = S1 / jnp.maximum(cnt, 1.0)
        var = (S2 - S1 * mean) / jnp.maximum(cnt - 1.0, 1.0)

        d_iota = jax.lax.broadcasted_iota(jnp.int32, M.shape, 1)
        row_min = jnp.min(M, axis=1, keepdims=True)
        am = jnp.min(jnp.where(M == row_min, d_iota, M.shape[1]), axis=1,
                     keepdims=True)

        iota_r = jax.lax.broadcasted_iota(jnp.int32, (T, T), 0)
        iota_c = jax.lax.broadcasted_iota(jnp.int32, (T, T), 1)
        eye = iota_r == iota_c
        valid = (~eye) & (am != am.reshape(1, T))
        scores = jnp.where(valid, var, 9999.0)

        s_min = jnp.min(scores, axis=1, keepdims=True)
        min_row = jnp.min(jnp.where(scores == s_min, iota_c, T), axis=1,
                          keepdims=True)
        onehot = min_row == iota_c

        h2 = jnp.sum(jnp.where(eye, R, 0.0), axis=1, keepdims=True)
        norm2 = jnp.maximum(h2 + h2.reshape(1, T) - 2.0 * R, 0.0)
        sel = jnp.sum(jnp.where(onehot, norm2, 0.0), axis=1)
        row_loss = jnp.sum(jnp.sqrt(sel))

        out_ref[...] = jnp.reshape(mse_a[0] + ROW_PENALTY * row_loss, (1, 1))


def kernel(X, H, C, M, T):
    Tn, D = X.shape
    nk = D // _DC
    chunk = pl.BlockSpec((Tn, _DC), lambda k: (0, k))
    whole = pl.BlockSpec((Tn, D), lambda k: (0, 0))
    out = pl.pallas_call(
        _loss_kernel,
        grid=(nk,),
        in_specs=[chunk, chunk, chunk, whole],
        out_specs=pl.BlockSpec((1, 1), lambda k: (0, 0)),
        out_shape=jax.ShapeDtypeStruct((1, 1), jnp.float32),
        scratch_shapes=[pltpu.VMEM((Tn, Tn), jnp.float32)] * 5
        + [pltpu.SMEM((1,), jnp.float32)],
        compiler_params=pltpu.CompilerParams(
            dimension_semantics=("arbitrary",),
        ),
    )(X, H, C, M)
    return out[0, 0]


# speedup vs baseline: 1.5931x; 1.2037x over previous
import functools

import jax
import jax.numpy as jnp
from jax.experimental import pallas as pl

ROW_PENALTY = 10.0


def _loss_kernel(x_ref, h_ref, c_ref, m_ref, out_ref):
    X = x_ref[...]
    H = h_ref[...]
    C = c_ref[...]
    M = m_ref[...]
    T = X.shape[0]

    f32 = jnp.float32
    mask = (M > 0).astype(f32)
    U = mask * H
    V = U * H

    dot_t = functools.partial(
        jax.lax.dot_general,
        dimension_numbers=(((1,), (1,)), ((), ())),
        preferred_element_type=f32,
    )

    cnt = dot_t(mask, mask)
    B = dot_t(U, mask)
    P = dot_t(V, mask)
    Q = dot_t(U, U)
    R = dot_t(H, H)

    S1 = B - B.T
    S2 = P + P.T - 2.0 * Q
    mean = S1 / jnp.maximum(cnt, 1.0)
    var = (S2 - S1 * mean) / jnp.maximum(cnt - 1.0, 1.0)

    d_iota = jax.lax.broadcasted_iota(jnp.int32, M.shape, 1)
    row_min = jnp.min(M, axis=1, keepdims=True)
    am = jnp.min(jnp.where(M == row_min, d_iota, M.shape[1]), axis=1,
                 keepdims=True)

    iota_r = jax.lax.broadcasted_iota(jnp.int32, (T, T), 0)
    iota_c = jax.lax.broadcasted_iota(jnp.int32, (T, T), 1)
    eye = iota_r == iota_c
    valid = (~eye) & (am != am.reshape(1, T))
    scores = jnp.where(valid, var, 9999.0)

    s_min = jnp.min(scores, axis=1, keepdims=True)
    min_row = jnp.min(jnp.where(scores == s_min, iota_c, T), axis=1,
                      keepdims=True)
    onehot = min_row == iota_c

    h2 = jnp.sum(jnp.where(eye, R, 0.0), axis=1, keepdims=True)
    norm2 = jnp.maximum(h2 + h2.reshape(1, T) - 2.0 * R, 0.0)
    sel = jnp.sum(jnp.where(onehot, norm2, 0.0), axis=1)
    row_loss = jnp.sum(jnp.sqrt(sel))

    resid = (X - H + C) * M
    mse = jnp.sum(resid * resid)

    out_ref[...] = jnp.reshape(mse + ROW_PENALTY * row_loss, (1, 1))


def kernel(X, H, C, M, T):
    out = pl.pallas_call(
        _loss_kernel,
        out_shape=jax.ShapeDtypeStruct((1, 1), jnp.float32),
    )(X, H, C, M)
    return out[0, 0]
